# Initial kernel scaffold; baseline (speedup 1.0000x reference)
import jax
import jax.numpy as jnp
from jax.experimental import pallas as pl


def kernel(pos, s, pair_rep, pair_mask, noise_level, W_rbf, W_n1, W_n2, W_mod, W_in2f, Wf1, Wf2, W_out1, W_out2, w_z):
    raise NotImplementedError("write your pallas kernel here")


# speedup vs baseline: 2.7488x; 2.7488x over previous
import jax
import jax.numpy as jnp
import numpy as np
from jax.experimental import pallas as pl
from jax.experimental.pallas import tpu as pltpu

N = 512
DIM_S = 128
DIM_Z = 64
N_RBF = 64
DIM_FILTER = 128
DIM_NOISE = 64
N_LAYERS = 3
R_MIN = 0.04
R_MAX = 10.0
EPS = 1e-5

BI = 128
BJ = 128

_LOG_RMIN = float(np.log(R_MIN))
_SIGMA = float((np.log(R_MAX) - np.log(R_MIN)) / (N_RBF - 1))
_INV_SIGMA = 1.0 / _SIGMA
_INV_FC = float(3.0 / R_MAX)

_HI = jax.lax.Precision.HIGHEST


def _silu(x):
    return x * jax.nn.sigmoid(x)


def _body(s_ref, pos_ref, z_ref, m_ref, nemb_ref, Wrbf_ref, Wmod_ref, Win_ref,
          Wf1_ref, Wf2_ref, Wo1_ref, Wo2_ref, wz_ref, out_ref,
          h_s, xf_s, mod_s, agg_s):
    l = pl.program_id(0)
    i = pl.program_id(1)
    j = pl.program_id(2)
    nj = pl.num_programs(2)

    @pl.when((l == 0) & (i == 0) & (j == 0))
    def _init():
        h_s[...] = s_ref[...]

    @pl.when((i == 0) & (j == 0))
    def _layer_prologue():
        mod = jnp.dot(nemb_ref[...], Wmod_ref[l], precision=_HI)
        mod_s[...] = mod
        h = h_s[...]
        mu = jnp.mean(h, axis=-1, keepdims=True)
        var = jnp.mean((h - mu) ** 2, axis=-1, keepdims=True)
        hn = (h - mu) * jax.lax.rsqrt(var + EPS)
        shift = mod[:, 0:DIM_S]
        scale = mod[:, DIM_S:2 * DIM_S]
        hn = hn * (1.0 + scale) + shift
        xf_s[...] = jnp.dot(hn, Win_ref[l], precision=_HI)

    pi = pos_ref[pl.ds(i * BI, BI), :]
    pj = pos_ref[pl.ds(j * BJ, BJ), :]
    rel = pi[:, None, :] - pj[None, :, :]
    d = jnp.sqrt(jnp.sum(rel * rel, axis=-1) + 1e-12)
    x = jnp.log(jnp.maximum(d, R_MIN))
    mu_k = _LOG_RMIN + _SIGMA * jax.lax.broadcasted_iota(
        jnp.int32, (1, 1, N_RBF), 2).astype(jnp.float32)
    t = (x[:, :, None] - mu_k) * _INV_SIGMA
    fcut = jnp.exp(-0.5 * (d * _INV_FC) ** 2)
    rbf = jnp.exp(-0.5 * t * t) * fcut[:, :, None]

    r = rbf.reshape(BI * BJ, N_RBF)
    r = _silu(jnp.dot(r, Wrbf_ref[...], precision=_HI))
    f = _silu(jnp.dot(r, Wf1_ref[l], precision=_HI))
    f = jnp.dot(f, Wf2_ref[l], precision=_HI)

    z = z_ref[...].reshape(BI * BJ, DIM_Z)
    eg = jax.nn.sigmoid(jnp.dot(z, wz_ref[l], precision=_HI))

    rows = i * BI + jax.lax.broadcasted_iota(jnp.int32, (BI, BJ), 0)
    cols = j * BJ + jax.lax.broadcasted_iota(jnp.int32, (BI, BJ), 1)
    mask = (m_ref[...] != 0.0) & (rows != cols)
    w = jnp.where(mask, eg.reshape(BI, BJ), 0.0)

    xfj = xf_s[pl.ds(j * BJ, BJ), :]
    msg = f.reshape(BI, BJ, DIM_FILTER) * xfj[None, :, :] * w[:, :, None]
    part = jnp.sum(msg, axis=1)

    @pl.when(j == 0)
    def _agg_init():
        agg_s[...] = part

    @pl.when(j > 0)
    def _agg_acc():
        agg_s[...] = agg_s[...] + part

    @pl.when(j == nj - 1)
    def _layer_epilogue():
        agg = agg_s[...]
        ds_ = jnp.dot(_silu(jnp.dot(agg, Wo1_ref[l], precision=_HI)),
                      Wo2_ref[l], precision=_HI)
        gate = mod_s[:, 2 * DIM_S:3 * DIM_S]
        hr = h_s[pl.ds(i * BI, BI), :] + gate * ds_
        h_s[pl.ds(i * BI, BI), :] = hr
        out_ref[...] = hr


def kernel(pos, s, pair_rep, pair_mask, noise_level, W_rbf, W_n1, W_n2,
           W_mod, W_in2f, Wf1, Wf2, W_out1, W_out2, w_z):
    noise = jnp.clip(noise_level, 1e-4, 1e2)
    lx = jnp.log(noise)
    nf = DIM_NOISE // 2
    freqs = jnp.pi * (2.0 ** jnp.arange(nf, dtype=jnp.float32))
    xph = lx[..., None] * freqs
    nemb = jnp.concatenate([jnp.sin(xph), jnp.cos(xph)], axis=-1)
    nemb = _silu(nemb @ W_n1)
    nemb = _silu(nemb @ W_n2)

    wz2 = w_z[:, :, 0]

    grid = (N_LAYERS, N // BI, N // BJ)
    out = pl.pallas_call(
        _body,
        grid=grid,
        in_specs=[
            pl.BlockSpec((N, DIM_S), lambda l, i, j: (0, 0)),
            pl.BlockSpec((N, 3), lambda l, i, j: (0, 0)),
            pl.BlockSpec((BI, BJ, DIM_Z), lambda l, i, j: (i, j, 0)),
            pl.BlockSpec((BI, BJ), lambda l, i, j: (i, j)),
            pl.BlockSpec((1, DIM_NOISE), lambda l, i, j: (0, 0)),
            pl.BlockSpec((N_RBF, N_RBF), lambda l, i, j: (0, 0)),
            pl.BlockSpec((N_LAYERS, DIM_NOISE, 3 * DIM_S),
                         lambda l, i, j: (0, 0, 0)),
            pl.BlockSpec((N_LAYERS, DIM_S, DIM_FILTER),
                         lambda l, i, j: (0, 0, 0)),
            pl.BlockSpec((N_LAYERS, N_RBF, DIM_FILTER),
                         lambda l, i, j: (0, 0, 0)),
            pl.BlockSpec((N_LAYERS, DIM_FILTER, DIM_FILTER),
                         lambda l, i, j: (0, 0, 0)),
            pl.BlockSpec((N_LAYERS, DIM_FILTER, DIM_S),
                         lambda l, i, j: (0, 0, 0)),
            pl.BlockSpec((N_LAYERS, DIM_S, DIM_S),
                         lambda l, i, j: (0, 0, 0)),
            pl.BlockSpec((N_LAYERS, DIM_Z), lambda l, i, j: (0, 0)),
        ],
        out_specs=pl.BlockSpec((BI, DIM_S), lambda l, i, j: (i, 0)),
        out_shape=jax.ShapeDtypeStruct((N, DIM_S), jnp.float32),
        scratch_shapes=[
            pltpu.VMEM((N, DIM_S), jnp.float32),
            pltpu.VMEM((N, DIM_S), jnp.float32),
            pltpu.VMEM((1, 3 * DIM_S), jnp.float32),
            pltpu.VMEM((BI, DIM_FILTER), jnp.float32),
        ],
    )(s, pos, pair_rep, pair_mask, nemb, W_rbf, W_mod, W_in2f, Wf1, Wf2,
      W_out1, W_out2, wz2)
    return out


# speedup vs baseline: 8.6646x; 3.1522x over previous
import jax
import jax.numpy as jnp
import numpy as np
from jax.experimental import pallas as pl
from jax.experimental.pallas import tpu as pltpu

N = 512
DIM_S = 128
DIM_Z = 64
N_RBF = 64
DIM_FILTER = 128
DIM_NOISE = 64
N_LAYERS = 3
R_MIN = 0.04
R_MAX = 10.0
EPS = 1e-5

BI = 128
BJ = 128

_LOG_RMIN = float(np.log(R_MIN))
_SIGMA = float((np.log(R_MAX) - np.log(R_MIN)) / (N_RBF - 1))
_INV_SIGMA = 1.0 / _SIGMA
_INV_FC = float(3.0 / R_MAX)

_HI = jax.lax.Precision.DEFAULT


def _silu(x):
    return x * jax.nn.sigmoid(x)


def _body(s_ref, pos_ref, z_ref, m_ref, nemb_ref, Wrbf_ref, Wmod_ref, Win_ref,
          Wf1_ref, Wf2_ref, Wo1_ref, Wo2_ref, wz_ref, out_ref,
          h_s, xf_s, mod_s, agg_s):
    l = pl.program_id(0)
    i = pl.program_id(1)
    j = pl.program_id(2)
    nj = pl.num_programs(2)

    @pl.when((l == 0) & (i == 0) & (j == 0))
    def _init():
        h_s[...] = s_ref[...]

    @pl.when((i == 0) & (j == 0))
    def _layer_prologue():
        mod = jnp.dot(nemb_ref[...], Wmod_ref[l], precision=_HI)
        mod_s[...] = mod
        h = h_s[...]
        mu = jnp.mean(h, axis=-1, keepdims=True)
        var = jnp.mean((h - mu) ** 2, axis=-1, keepdims=True)
        hn = (h - mu) * jax.lax.rsqrt(var + EPS)
        shift = mod[:, 0:DIM_S]
        scale = mod[:, DIM_S:2 * DIM_S]
        hn = hn * (1.0 + scale) + shift
        xf_s[...] = jnp.dot(hn, Win_ref[l], precision=_HI)

    pi = pos_ref[pl.ds(i * BI, BI), :]
    pj = pos_ref[pl.ds(j * BJ, BJ), :]
    rel = pi[:, None, :] - pj[None, :, :]
    d = jnp.sqrt(jnp.sum(rel * rel, axis=-1) + 1e-12)
    x = jnp.log(jnp.maximum(d, R_MIN))
    mu_k = _LOG_RMIN + _SIGMA * jax.lax.broadcasted_iota(
        jnp.int32, (1, 1, N_RBF), 2).astype(jnp.float32)
    t = (x[:, :, None] - mu_k) * _INV_SIGMA
    fcut = jnp.exp(-0.5 * (d * _INV_FC) ** 2)
    rbf = jnp.exp(-0.5 * t * t) * fcut[:, :, None]

    r = rbf.reshape(BI * BJ, N_RBF)
    r = _silu(jnp.dot(r, Wrbf_ref[...], precision=_HI))
    f = _silu(jnp.dot(r, Wf1_ref[l], precision=_HI))
    f = jnp.dot(f, Wf2_ref[l], precision=_HI)

    z = z_ref[...].reshape(BI * BJ, DIM_Z)
    eg = jax.nn.sigmoid(jnp.dot(z, wz_ref[l], precision=_HI))

    rows = i * BI + jax.lax.broadcasted_iota(jnp.int32, (BI, BJ), 0)
    cols = j * BJ + jax.lax.broadcasted_iota(jnp.int32, (BI, BJ), 1)
    mask = (m_ref[...] != 0.0) & (rows != cols)
    w = jnp.where(mask, eg.reshape(BI, BJ), 0.0)

    xfj = xf_s[pl.ds(j * BJ, BJ), :]
    msg = f.reshape(BI, BJ, DIM_FILTER) * xfj[None, :, :] * w[:, :, None]
    part = jnp.sum(msg, axis=1)

    @pl.when(j == 0)
    def _agg_init():
        agg_s[...] = part

    @pl.when(j > 0)
    def _agg_acc():
        agg_s[...] = agg_s[...] + part

    @pl.when(j == nj - 1)
    def _layer_epilogue():
        agg = agg_s[...]
        ds_ = jnp.dot(_silu(jnp.dot(agg, Wo1_ref[l], precision=_HI)),
                      Wo2_ref[l], precision=_HI)
        gate = mod_s[:, 2 * DIM_S:3 * DIM_S]
        hr = h_s[pl.ds(i * BI, BI), :] + gate * ds_
        h_s[pl.ds(i * BI, BI), :] = hr
        out_ref[...] = hr


def kernel(pos, s, pair_rep, pair_mask, noise_level, W_rbf, W_n1, W_n2,
           W_mod, W_in2f, Wf1, Wf2, W_out1, W_out2, w_z):
    noise = jnp.clip(noise_level, 1e-4, 1e2)
    lx = jnp.log(noise)
    nf = DIM_NOISE // 2
    freqs = jnp.pi * (2.0 ** jnp.arange(nf, dtype=jnp.float32))
    xph = lx[..., None] * freqs
    nemb = jnp.concatenate([jnp.sin(xph), jnp.cos(xph)], axis=-1)
    nemb = _silu(nemb @ W_n1)
    nemb = _silu(nemb @ W_n2)

    wz2 = w_z[:, :, 0]

    grid = (N_LAYERS, N // BI, N // BJ)
    out = pl.pallas_call(
        _body,
        grid=grid,
        in_specs=[
            pl.BlockSpec((N, DIM_S), lambda l, i, j: (0, 0)),
            pl.BlockSpec((N, 3), lambda l, i, j: (0, 0)),
            pl.BlockSpec((BI, BJ, DIM_Z), lambda l, i, j: (i, j, 0)),
            pl.BlockSpec((BI, BJ), lambda l, i, j: (i, j)),
            pl.BlockSpec((1, DIM_NOISE), lambda l, i, j: (0, 0)),
            pl.BlockSpec((N_RBF, N_RBF), lambda l, i, j: (0, 0)),
            pl.BlockSpec((N_LAYERS, DIM_NOISE, 3 * DIM_S),
                         lambda l, i, j: (0, 0, 0)),
            pl.BlockSpec((N_LAYERS, DIM_S, DIM_FILTER),
                         lambda l, i, j: (0, 0, 0)),
            pl.BlockSpec((N_LAYERS, N_RBF, DIM_FILTER),
                         lambda l, i, j: (0, 0, 0)),
            pl.BlockSpec((N_LAYERS, DIM_FILTER, DIM_FILTER),
                         lambda l, i, j: (0, 0, 0)),
            pl.BlockSpec((N_LAYERS, DIM_FILTER, DIM_S),
                         lambda l, i, j: (0, 0, 0)),
            pl.BlockSpec((N_LAYERS, DIM_S, DIM_S),
                         lambda l, i, j: (0, 0, 0)),
            pl.BlockSpec((N_LAYERS, DIM_Z), lambda l, i, j: (0, 0)),
        ],
        out_specs=pl.BlockSpec((BI, DIM_S), lambda l, i, j: (i, 0)),
        out_shape=jax.ShapeDtypeStruct((N, DIM_S), jnp.float32),
        scratch_shapes=[
            pltpu.VMEM((N, DIM_S), jnp.float32),
            pltpu.VMEM((N, DIM_S), jnp.float32),
            pltpu.VMEM((1, 3 * DIM_S), jnp.float32),
            pltpu.VMEM((BI, DIM_FILTER), jnp.float32),
        ],
    )(s, pos, pair_rep, pair_mask, nemb, W_rbf, W_mod, W_in2f, Wf1, Wf2,
      W_out1, W_out2, wz2)
    return out


# speedup vs baseline: 11.0935x; 1.2803x over previous
import jax
import jax.numpy as jnp
import numpy as np
from jax.experimental import pallas as pl
from jax.experimental.pallas import tpu as pltpu

N = 512
DIM_S = 128
DIM_Z = 64
N_RBF = 64
DIM_FILTER = 128
DIM_NOISE = 64
N_LAYERS = 3
R_MIN = 0.04
R_MAX = 10.0
EPS = 1e-5

B = 128
T = N // B
TT = T * (T + 1) // 2

_LOG_RMIN = float(np.log(R_MIN))
_SIGMA = float((np.log(R_MAX) - np.log(R_MIN)) / (N_RBF - 1))
_INV_SIGMA = 1.0 / _SIGMA
_INV_FC = float(3.0 / R_MAX)


def _silu(x):
    return x * jax.nn.sigmoid(x)


def _body(ta_ref, tb_ref, s_ref, pos_ref, z_ab_ref, z_ba_ref, m_ab_ref,
          m_ba_ref, nemb_ref, Wrbf_ref, Wmod_ref, Win_ref, Wf1_ref, Wf2_ref,
          Wo1_ref, Wo2_ref, wz_ref, out_ref, h_s, xf_s, mod_s, agg_s):
    l = pl.program_id(0)
    t = pl.program_id(1)
    a = ta_ref[t]
    b = tb_ref[t]

    @pl.when((l == 0) & (t == 0))
    def _init():
        h_s[...] = s_ref[...]

    @pl.when(t == 0)
    def _layer_prologue():
        mod = jnp.dot(nemb_ref[...], Wmod_ref[l])
        mod_s[...] = mod
        h = h_s[...]
        mu = jnp.mean(h, axis=-1, keepdims=True)
        var = jnp.mean((h - mu) ** 2, axis=-1, keepdims=True)
        hn = (h - mu) * jax.lax.rsqrt(var + EPS)
        shift = mod[:, 0:DIM_S]
        scale = mod[:, DIM_S:2 * DIM_S]
        hn = hn * (1.0 + scale) + shift
        xf_s[...] = jnp.dot(hn, Win_ref[l])
        agg_s[...] = jnp.zeros_like(agg_s)

    pa = pos_ref[pl.ds(a * B, B), :]
    pb = pos_ref[pl.ds(b * B, B), :]
    rel = pa[:, None, :] - pb[None, :, :]
    d = jnp.sqrt(jnp.sum(rel * rel, axis=-1) + 1e-12)
    x = jnp.log(jnp.maximum(d, R_MIN))
    mu_k = _LOG_RMIN + _SIGMA * jax.lax.broadcasted_iota(
        jnp.int32, (1, 1, N_RBF), 2).astype(jnp.float32)
    tt = (x[:, :, None] - mu_k) * _INV_SIGMA
    fcut = jnp.exp(-0.5 * (d * _INV_FC) ** 2)
    rbf = jnp.exp(-0.5 * tt * tt) * fcut[:, :, None]

    r = rbf.reshape(B * B, N_RBF)
    r = _silu(jnp.dot(r, Wrbf_ref[...]))
    u = _silu(jnp.dot(r, Wf1_ref[l]))
    fil = jnp.dot(u, Wf2_ref[l]).reshape(B, B, DIM_FILTER)

    rows = a * B + jax.lax.broadcasted_iota(jnp.int32, (B, B), 0)
    cols = b * B + jax.lax.broadcasted_iota(jnp.int32, (B, B), 1)

    eg = jax.nn.sigmoid(
        jnp.dot(z_ab_ref[...].reshape(B * B, DIM_Z), wz_ref[l]))
    mask = (m_ab_ref[...] != 0.0) & (rows != cols)
    w_ab = jnp.where(mask, eg.reshape(B, B), 0.0)
    xf_b = xf_s[pl.ds(b * B, B), :]
    part_a = jnp.sum(fil * w_ab[:, :, None] * xf_b[None, :, :], axis=1)
    agg_s[pl.ds(a * B, B), :] += part_a

    @pl.when(a != b)
    def _mirror():
        eg_m = jax.nn.sigmoid(
            jnp.dot(z_ba_ref[...].reshape(B * B, DIM_Z), wz_ref[l]))
        mask_m = m_ba_ref[...] != 0.0
        w_ba = jnp.where(mask_m, eg_m.reshape(B, B), 0.0)
        w_t = w_ba.T
        xf_a = xf_s[pl.ds(a * B, B), :]
        part_b = jnp.sum(fil * w_t[:, :, None] * xf_a[:, None, :], axis=0)
        agg_s[pl.ds(b * B, B), :] += part_b

    @pl.when(t == TT - 1)
    def _layer_epilogue():
        agg = agg_s[...]
        ds_ = jnp.dot(_silu(jnp.dot(agg, Wo1_ref[l])), Wo2_ref[l])
        gate = mod_s[:, 2 * DIM_S:3 * DIM_S]
        h = h_s[...] + gate * ds_
        h_s[...] = h
        out_ref[...] = h


def kernel(pos, s, pair_rep, pair_mask, noise_level, W_rbf, W_n1, W_n2,
           W_mod, W_in2f, Wf1, Wf2, W_out1, W_out2, w_z):
    noise = jnp.clip(noise_level, 1e-4, 1e2)
    lx = jnp.log(noise)
    nf = DIM_NOISE // 2
    freqs = jnp.pi * (2.0 ** jnp.arange(nf, dtype=jnp.float32))
    xph = lx[..., None] * freqs
    nemb = jnp.concatenate([jnp.sin(xph), jnp.cos(xph)], axis=-1)
    nemb = _silu(nemb @ W_n1)
    nemb = _silu(nemb @ W_n2)

    wz2 = w_z[:, :, 0]

    pairs = [(aa, bb) for aa in range(T) for bb in range(aa, T)]
    ta = jnp.asarray([p[0] for p in pairs], dtype=jnp.int32)
    tb = jnp.asarray([p[1] for p in pairs], dtype=jnp.int32)

    grid_spec = pltpu.PrefetchScalarGridSpec(
        num_scalar_prefetch=2,
        grid=(N_LAYERS, TT),
        in_specs=[
            pl.BlockSpec((N, DIM_S), lambda l, t, ta_, tb_: (0, 0)),
            pl.BlockSpec((N, 3), lambda l, t, ta_, tb_: (0, 0)),
            pl.BlockSpec((B, B, DIM_Z),
                         lambda l, t, ta_, tb_: (ta_[t], tb_[t], 0)),
            pl.BlockSpec((B, B, DIM_Z),
                         lambda l, t, ta_, tb_: (tb_[t], ta_[t], 0)),
            pl.BlockSpec((B, B),
                         lambda l, t, ta_, tb_: (ta_[t], tb_[t])),
            pl.BlockSpec((B, B),
                         lambda l, t, ta_, tb_: (tb_[t], ta_[t])),
            pl.BlockSpec((1, DIM_NOISE), lambda l, t, ta_, tb_: (0, 0)),
            pl.BlockSpec((N_RBF, N_RBF), lambda l, t, ta_, tb_: (0, 0)),
            pl.BlockSpec((N_LAYERS, DIM_NOISE, 3 * DIM_S),
                         lambda l, t, ta_, tb_: (0, 0, 0)),
            pl.BlockSpec((N_LAYERS, DIM_S, DIM_FILTER),
                         lambda l, t, ta_, tb_: (0, 0, 0)),
            pl.BlockSpec((N_LAYERS, N_RBF, DIM_FILTER),
                         lambda l, t, ta_, tb_: (0, 0, 0)),
            pl.BlockSpec((N_LAYERS, DIM_FILTER, DIM_FILTER),
                         lambda l, t, ta_, tb_: (0, 0, 0)),
            pl.BlockSpec((N_LAYERS, DIM_FILTER, DIM_S),
                         lambda l, t, ta_, tb_: (0, 0, 0)),
            pl.BlockSpec((N_LAYERS, DIM_S, DIM_S),
                         lambda l, t, ta_, tb_: (0, 0, 0)),
            pl.BlockSpec((N_LAYERS, DIM_Z),
                         lambda l, t, ta_, tb_: (0, 0)),
        ],
        out_specs=pl.BlockSpec((N, DIM_S), lambda l, t, ta_, tb_: (0, 0)),
        scratch_shapes=[
            pltpu.VMEM((N, DIM_S), jnp.float32),
            pltpu.VMEM((N, DIM_S), jnp.float32),
            pltpu.VMEM((1, 3 * DIM_S), jnp.float32),
            pltpu.VMEM((N, DIM_FILTER), jnp.float32),
        ],
    )
    out = pl.pallas_call(
        _body,
        grid_spec=grid_spec,
        out_shape=jax.ShapeDtypeStruct((N, DIM_S), jnp.float32),
    )(ta, tb, s, pos, pair_rep, pair_rep, pair_mask, pair_mask, nemb, W_rbf,
      W_mod, W_in2f, Wf1, Wf2, W_out1, W_out2, wz2)
    return out


# speedup vs baseline: 13.3665x; 1.2049x over previous
import jax
import jax.numpy as jnp
import numpy as np
from jax.experimental import pallas as pl
from jax.experimental.pallas import tpu as pltpu

N = 512
DIM_S = 128
DIM_Z = 64
N_RBF = 64
DIM_FILTER = 128
DIM_NOISE = 64
N_LAYERS = 3
R_MIN = 0.04
R_MAX = 10.0
EPS = 1e-5

B = 64
T = N // B
TT = T * (T + 1) // 2

_LOG_RMIN = float(np.log(R_MIN))
_SIGMA = float((np.log(R_MAX) - np.log(R_MIN)) / (N_RBF - 1))
_INV_SIGMA = 1.0 / _SIGMA
_INV_FC = float(3.0 / R_MAX)


def _silu(x):
    return x * jax.nn.sigmoid(x)


def _rbf_body(ta_ref, tb_ref, pos_ref, Wrbf_ref, r_ref):
    t = pl.program_id(0)
    a = ta_ref[t]
    b = tb_ref[t]
    pa = pos_ref[pl.ds(a * B, B), :]
    pb = pos_ref[pl.ds(b * B, B), :]
    rel = pa[:, None, :] - pb[None, :, :]
    d = jnp.sqrt(jnp.sum(rel * rel, axis=-1) + 1e-12)
    x = jnp.log(jnp.maximum(d, R_MIN))
    mu_k = _LOG_RMIN + _SIGMA * jax.lax.broadcasted_iota(
        jnp.int32, (1, 1, N_RBF), 2).astype(jnp.float32)
    tt = (x[:, :, None] - mu_k) * _INV_SIGMA
    fcut = jnp.exp(-0.5 * (d * _INV_FC) ** 2)
    rbf = jnp.exp(-0.5 * tt * tt) * fcut[:, :, None]
    r = _silu(jnp.dot(rbf.reshape(B * B, N_RBF), Wrbf_ref[...]))
    r_ref[...] = r.reshape(1, B, B, N_RBF)


def _main_body(ta_ref, tb_ref, s_ref, r_ref, z_ab_ref, z_ba_ref, m_ab_ref,
               m_ba_ref, nemb_ref, Wmod_ref, Win_ref, Wf1_ref, Wf2_ref,
               Wo1_ref, Wo2_ref, wz_ref, out_ref, h_s, xf_s, mod_s, agg_s):
    l = pl.program_id(0)
    t = pl.program_id(1)
    a = ta_ref[t]
    b = tb_ref[t]

    @pl.when((l == 0) & (t == 0))
    def _init():
        h_s[...] = s_ref[...]

    @pl.when(t == 0)
    def _layer_prologue():
        mod = jnp.dot(nemb_ref[...], Wmod_ref[l])
        mod_s[...] = mod
        h = h_s[...]
        mu = jnp.mean(h, axis=-1, keepdims=True)
        var = jnp.mean((h - mu) ** 2, axis=-1, keepdims=True)
        hn = (h - mu) * jax.lax.rsqrt(var + EPS)
        shift = mod[:, 0:DIM_S]
        scale = mod[:, DIM_S:2 * DIM_S]
        hn = hn * (1.0 + scale) + shift
        xf_s[...] = jnp.dot(hn, Win_ref[l])
        agg_s[...] = jnp.zeros_like(agg_s)

    r = r_ref[...].reshape(B * B, N_RBF)
    u = _silu(jnp.dot(r, Wf1_ref[l]))
    fil = jnp.dot(u, Wf2_ref[l]).reshape(B, B, DIM_FILTER)

    rows = a * B + jax.lax.broadcasted_iota(jnp.int32, (B, B), 0)
    cols = b * B + jax.lax.broadcasted_iota(jnp.int32, (B, B), 1)

    eg = jax.nn.sigmoid(
        jnp.dot(z_ab_ref[...].reshape(B * B, DIM_Z), wz_ref[l]))
    mask = (m_ab_ref[...].reshape(B, B) != 0.0) & (rows != cols)
    w_ab = jnp.where(mask, eg.reshape(B, B), 0.0)
    xf_b = xf_s[pl.ds(b * B, B), :]
    part_a = jnp.sum(fil * w_ab[:, :, None] * xf_b[None, :, :], axis=1)
    agg_s[pl.ds(a * B, B), :] += part_a

    @pl.when(a != b)
    def _mirror():
        eg_m = jax.nn.sigmoid(
            jnp.dot(z_ba_ref[...].reshape(B * B, DIM_Z), wz_ref[l]))
        mask_m = m_ba_ref[...].reshape(B, B) != 0.0
        w_ba = jnp.where(mask_m, eg_m.reshape(B, B), 0.0)
        w_t = w_ba.T
        xf_a = xf_s[pl.ds(a * B, B), :]
        part_b = jnp.sum(fil * w_t[:, :, None] * xf_a[:, None, :], axis=0)
        agg_s[pl.ds(b * B, B), :] += part_b

    @pl.when(t == TT - 1)
    def _layer_epilogue():
        agg = agg_s[...]
        ds_ = jnp.dot(_silu(jnp.dot(agg, Wo1_ref[l])), Wo2_ref[l])
        gate = mod_s[:, 2 * DIM_S:3 * DIM_S]
        h = h_s[...] + gate * ds_
        h_s[...] = h
        out_ref[...] = h


def kernel(pos, s, pair_rep, pair_mask, noise_level, W_rbf, W_n1, W_n2,
           W_mod, W_in2f, Wf1, Wf2, W_out1, W_out2, w_z):
    noise = jnp.clip(noise_level, 1e-4, 1e2)
    lx = jnp.log(noise)
    nf = DIM_NOISE // 2
    freqs = jnp.pi * (2.0 ** jnp.arange(nf, dtype=jnp.float32))
    xph = lx[..., None] * freqs
    nemb = jnp.concatenate([jnp.sin(xph), jnp.cos(xph)], axis=-1)
    nemb = _silu(nemb @ W_n1)
    nemb = _silu(nemb @ W_n2)

    wz2 = w_z[:, :, 0]
    pm4 = pair_mask.reshape(T, B, T, B).transpose(0, 2, 1, 3)

    pairs = [(aa, bb) for aa in range(T) for bb in range(aa, T)]
    ta = jnp.asarray([p[0] for p in pairs], dtype=jnp.int32)
    tb = jnp.asarray([p[1] for p in pairs], dtype=jnp.int32)

    rbf_spec = pltpu.PrefetchScalarGridSpec(
        num_scalar_prefetch=2,
        grid=(TT,),
        in_specs=[
            pl.BlockSpec((N, 3), lambda t, ta_, tb_: (0, 0)),
            pl.BlockSpec((N_RBF, N_RBF), lambda t, ta_, tb_: (0, 0)),
        ],
        out_specs=pl.BlockSpec((1, B, B, N_RBF),
                               lambda t, ta_, tb_: (t, 0, 0, 0)),
    )
    r_all = pl.pallas_call(
        _rbf_body,
        grid_spec=rbf_spec,
        out_shape=jax.ShapeDtypeStruct((TT, B, B, N_RBF), jnp.float32),
    )(ta, tb, pos, W_rbf)

    grid_spec = pltpu.PrefetchScalarGridSpec(
        num_scalar_prefetch=2,
        grid=(N_LAYERS, TT),
        in_specs=[
            pl.BlockSpec((N, DIM_S), lambda l, t, ta_, tb_: (0, 0)),
            pl.BlockSpec((1, B, B, N_RBF),
                         lambda l, t, ta_, tb_: (t, 0, 0, 0)),
            pl.BlockSpec((B, B, DIM_Z),
                         lambda l, t, ta_, tb_: (ta_[t], tb_[t], 0)),
            pl.BlockSpec((B, B, DIM_Z),
                         lambda l, t, ta_, tb_: (tb_[t], ta_[t], 0)),
            pl.BlockSpec((1, 1, B, B),
                         lambda l, t, ta_, tb_: (ta_[t], tb_[t], 0, 0)),
            pl.BlockSpec((1, 1, B, B),
                         lambda l, t, ta_, tb_: (tb_[t], ta_[t], 0, 0)),
            pl.BlockSpec((1, DIM_NOISE), lambda l, t, ta_, tb_: (0, 0)),
            pl.BlockSpec((N_LAYERS, DIM_NOISE, 3 * DIM_S),
                         lambda l, t, ta_, tb_: (0, 0, 0)),
            pl.BlockSpec((N_LAYERS, DIM_S, DIM_FILTER),
                         lambda l, t, ta_, tb_: (0, 0, 0)),
            pl.BlockSpec((N_LAYERS, N_RBF, DIM_FILTER),
                         lambda l, t, ta_, tb_: (0, 0, 0)),
            pl.BlockSpec((N_LAYERS, DIM_FILTER, DIM_FILTER),
                         lambda l, t, ta_, tb_: (0, 0, 0)),
            pl.BlockSpec((N_LAYERS, DIM_FILTER, DIM_S),
                         lambda l, t, ta_, tb_: (0, 0, 0)),
            pl.BlockSpec((N_LAYERS, DIM_S, DIM_S),
                         lambda l, t, ta_, tb_: (0, 0, 0)),
            pl.BlockSpec((N_LAYERS, DIM_Z),
                         lambda l, t, ta_, tb_: (0, 0)),
        ],
        out_specs=pl.BlockSpec((N, DIM_S), lambda l, t, ta_, tb_: (0, 0)),
        scratch_shapes=[
            pltpu.VMEM((N, DIM_S), jnp.float32),
            pltpu.VMEM((N, DIM_S), jnp.float32),
            pltpu.VMEM((1, 3 * DIM_S), jnp.float32),
            pltpu.VMEM((N, DIM_FILTER), jnp.float32),
        ],
    )
    out = pl.pallas_call(
        _main_body,
        grid_spec=grid_spec,
        out_shape=jax.ShapeDtypeStruct((N, DIM_S), jnp.float32),
    )(ta, tb, s, r_all, pair_rep, pair_rep, pm4, pm4, nemb,
      W_mod, W_in2f, Wf1, Wf2, W_out1, W_out2, wz2)
    return out
